# baseline (speedup 1.0000x reference)
import functools

import numpy as np
import jax
import jax.numpy as jnp
from jax import lax
from jax.experimental import pallas as pl
from jax.experimental.pallas import tpu as pltpu
from jax.experimental.pallas import tpu_sc as plsc

_ELEC_ROWS = np.array(
    [
        [0, 0, 0, 0, 0, 0, 0, 0, 0, 0, 0, 0, 0, 0, 0, 0],
        [0, 1, 0, 0, 0, 0, 0, 0, 0, 0, 0, 0, 0, 0, 0, 0],
        [2, 0, 0, 0, 0, 0, 0, 0, 0, 0, 0, 0, 0, 0, 0, 0],
        [2, 0, 0, 1, 0, 0, 0, 0, 0, 0, 0, 0, 0, 0, 0, 0],
        [2, 0, 2, 0, 0, 0, 0, 0, 0, 0, 0, 0, 0, 0, 0, 0],
        [2, 0, 2, 0, 0, 1, 0, 0, 0, 0, 0, 0, 0, 0, 0, 0],
        [2, 0, 2, 0, 0, 2, 0, 0, 0, 0, 0, 0, 0, 0, 0, 0],
        [2, 0, 2, 0, 0, 3, 0, 0, 0, 0, 0, 0, 0, 0, 0, 0],
        [2, 0, 2, 0, 2, 2, 0, 0, 0, 0, 0, 0, 0, 0, 0, 0],
        [2, 0, 2, 0, 4, 1, 0, 0, 0, 0, 0, 0, 0, 0, 0, 0],
    ],
    dtype=np.float32,
)
_ELEC_NORM = _ELEC_ROWS / _ELEC_ROWS.max()
_VPAD = 16
_ELEC_PAD = np.zeros((_VPAD, 16), dtype=np.float32)
_ELEC_PAD[:10] = _ELEC_NORM


def _fused_table_body(elec_ref, nuc_ref, elec_w_ref, ls_w_ref, ls_b_ref, out_ref):
    elec_emb = jnp.dot(elec_ref[...], elec_w_ref[...],
                       preferred_element_type=jnp.float32)
    h = nuc_ref[...] + elec_emb
    h = jnp.dot(h, ls_w_ref[...], preferred_element_type=jnp.float32)
    h = h + ls_b_ref[...]
    out_ref[...] = h * jax.nn.sigmoid(h)


def _compute_fused_table(nuclare_table, elec_W, ls_W, ls_b):
    vocab, num_features = nuclare_table.shape
    nuc_pad = jnp.zeros((_VPAD, num_features), jnp.float32).at[:vocab].set(nuclare_table)
    elec_pad = jnp.asarray(_ELEC_PAD)
    return pl.pallas_call(
        _fused_table_body,
        out_shape=jax.ShapeDtypeStruct((_VPAD, num_features), jnp.float32),
    )(elec_pad, nuc_pad, elec_W, ls_W, ls_b.reshape(1, num_features))


_NBUF = 2
_GPB = 2


def _make_sc_gather(B, D, nc, ns, chunk=128):
    nw = nc * ns
    b_per_w = B // nw
    n_chunks = b_per_w // chunk
    bufrows = _GPB * chunk
    n_bufs = n_chunks // _GPB
    n_groups = n_bufs // _NBUF
    assert n_chunks % (_GPB * _NBUF) == 0
    mesh = plsc.VectorSubcoreMesh(core_axis_name="c", subcore_axis_name="s")

    @functools.partial(
        pl.kernel,
        mesh=mesh,
        out_type=jax.ShapeDtypeStruct((B, D), jnp.float32),
        scratch_types=(
            [pltpu.VMEM((n_chunks, chunk), jnp.int32)]
            + [pltpu.VMEM((bufrows, D), jnp.float32) for _ in range(_NBUF)]
            + [pltpu.VMEM_SHARED((ns, _VPAD, D), jnp.float32)]
            + [pltpu.SemaphoreType.DMA for _ in range(2 * _NBUF)]
        ),
    )
    def gather_kernel(z_hbm, table_hbm, out_hbm, idx_all, *bufs_and_sems):
        rows = bufs_and_sems[:_NBUF]
        spm = bufs_and_sems[_NBUF]
        gsems = bufs_and_sems[_NBUF + 1:2 * _NBUF + 1]
        wsems = bufs_and_sems[2 * _NBUF + 1:]
        sid = lax.axis_index("s")
        wid = sid * nc + lax.axis_index("c")
        chunk0 = wid * n_chunks
        pltpu.sync_copy(table_hbm, rows[0].at[pl.ds(0, _VPAD)])
        pltpu.sync_copy(rows[0].at[pl.ds(0, _VPAD)], spm.at[sid])
        tbl = spm.at[sid]

        pltpu.sync_copy(z_hbm.at[pl.ds(chunk0, n_chunks)], idx_all)

        def gather_start(b, p):
            for u in range(_GPB):
                pltpu.async_copy(
                    tbl.at[idx_all.at[p * _GPB + u]],
                    rows[b].at[pl.ds(u * chunk, chunk)],
                    gsems[b])

        def gather_wait(b):
            for u in range(_GPB):
                pltpu.make_async_copy(
                    tbl.at[idx_all.at[0]],
                    rows[b].at[pl.ds(u * chunk, chunk)],
                    gsems[b]).wait()

        def write_wait(b):
            pltpu.make_async_copy(
                rows[b], out_hbm.at[pl.ds(0, bufrows)], wsems[b]).wait()

        for b in range(_NBUF):
            gather_start(b, b)

        def group(g, carry):
            p0 = g * _NBUF
            for b in range(_NBUF):
                gather_wait(b)
                pltpu.async_copy(
                    rows[b],
                    out_hbm.at[pl.ds(chunk0 * chunk + (p0 + b) * bufrows,
                                     bufrows)],
                    wsems[b])
            for b in range(_NBUF):
                pn = p0 + b + _NBUF

                @pl.when(pn < n_bufs)
                def _():
                    write_wait(b)
                    gather_start(b, pn)

            return carry

        lax.fori_loop(0, n_groups, group, 0)

        for b in range(_NBUF):
            write_wait(b)

    return gather_kernel


def kernel(z, nuclare_table, elec_W, ls_W, ls_b):
    Bz, L = z.shape
    num_features = nuclare_table.shape[1]
    B = Bz * L

    fused = _compute_fused_table(nuclare_table, elec_W, ls_W, ls_b)

    info = plsc.get_sparse_core_info()
    gather_kernel = _make_sc_gather(B, num_features, info.num_cores,
                                    info.num_subcores)
    z2d = z.reshape(B // 128, 128).astype(jnp.int32)
    out = gather_kernel(z2d, fused)
    return out.reshape(Bz, L, num_features)


# speedup vs baseline: 1.3898x; 1.3898x over previous
import functools

import numpy as np
import jax
import jax.numpy as jnp
from jax import lax
from jax.experimental import pallas as pl
from jax.experimental.pallas import tpu as pltpu
from jax.experimental.pallas import tpu_sc as plsc

_ELEC_ROWS = np.array(
    [
        [0, 0, 0, 0, 0, 0, 0, 0, 0, 0, 0, 0, 0, 0, 0, 0],
        [0, 1, 0, 0, 0, 0, 0, 0, 0, 0, 0, 0, 0, 0, 0, 0],
        [2, 0, 0, 0, 0, 0, 0, 0, 0, 0, 0, 0, 0, 0, 0, 0],
        [2, 0, 0, 1, 0, 0, 0, 0, 0, 0, 0, 0, 0, 0, 0, 0],
        [2, 0, 2, 0, 0, 0, 0, 0, 0, 0, 0, 0, 0, 0, 0, 0],
        [2, 0, 2, 0, 0, 1, 0, 0, 0, 0, 0, 0, 0, 0, 0, 0],
        [2, 0, 2, 0, 0, 2, 0, 0, 0, 0, 0, 0, 0, 0, 0, 0],
        [2, 0, 2, 0, 0, 3, 0, 0, 0, 0, 0, 0, 0, 0, 0, 0],
        [2, 0, 2, 0, 2, 2, 0, 0, 0, 0, 0, 0, 0, 0, 0, 0],
        [2, 0, 2, 0, 4, 1, 0, 0, 0, 0, 0, 0, 0, 0, 0, 0],
    ],
    dtype=np.float32,
)
_ELEC_NORM = _ELEC_ROWS / _ELEC_ROWS.max()
_VPAD = 16
_ELEC_PAD = np.zeros((_VPAD, 16), dtype=np.float32)
_ELEC_PAD[:10] = _ELEC_NORM


def _fused_table_body(elec_ref, nuc_ref, elec_w_ref, ls_w_ref, ls_b_ref, out_ref):
    elec_emb = jnp.dot(elec_ref[...], elec_w_ref[...],
                       preferred_element_type=jnp.float32)
    h = nuc_ref[...] + elec_emb
    h = jnp.dot(h, ls_w_ref[...], preferred_element_type=jnp.float32)
    h = h + ls_b_ref[...]
    out_ref[...] = h * jax.nn.sigmoid(h)


def _compute_fused_table(nuclare_table, elec_W, ls_W, ls_b):
    vocab, num_features = nuclare_table.shape
    nuc_pad = jnp.zeros((_VPAD, num_features), jnp.float32).at[:vocab].set(nuclare_table)
    elec_pad = jnp.asarray(_ELEC_PAD)
    return pl.pallas_call(
        _fused_table_body,
        out_shape=jax.ShapeDtypeStruct((_VPAD, num_features), jnp.float32),
    )(elec_pad, nuc_pad, elec_W, ls_W, ls_b.reshape(1, num_features))


_NBUF = 5


def _make_sc_gather(B, D, nc, ns, chunk=128):
    nw = nc * ns
    b_per_w = B // nw
    n_chunks = b_per_w // chunk
    n_groups = n_chunks // _NBUF
    assert n_chunks % _NBUF == 0
    mesh = plsc.VectorSubcoreMesh(core_axis_name="c", subcore_axis_name="s")

    @functools.partial(
        pl.kernel,
        mesh=mesh,
        out_type=jax.ShapeDtypeStruct((B, D), jnp.float32),
        scratch_types=(
            [pltpu.VMEM((n_chunks, chunk), jnp.int32)]
            + [pltpu.VMEM((chunk, D), jnp.float32) for _ in range(_NBUF)]
            + [pltpu.VMEM_SHARED((ns, _VPAD, D), jnp.float32)]
            + [pltpu.SemaphoreType.DMA for _ in range(2 * _NBUF)]
        ),
    )
    def gather_kernel(z_hbm, table_hbm, out_hbm, idx_all, *bufs_and_sems):
        rows = bufs_and_sems[:_NBUF]
        spm = bufs_and_sems[_NBUF]
        gsems = bufs_and_sems[_NBUF + 1:2 * _NBUF + 1]
        wsems = bufs_and_sems[2 * _NBUF + 1:]
        sid = lax.axis_index("s")
        wid = sid * nc + lax.axis_index("c")
        chunk0 = wid * n_chunks
        pltpu.sync_copy(table_hbm, rows[0].at[pl.ds(0, _VPAD)])
        pltpu.sync_copy(rows[0].at[pl.ds(0, _VPAD)], spm.at[sid])
        tbl = spm.at[sid]

        pltpu.sync_copy(z_hbm.at[pl.ds(chunk0, n_chunks)], idx_all)

        def gather_wait(b):
            pltpu.make_async_copy(
                tbl.at[idx_all.at[0]], rows[b], gsems[b]).wait()

        def write_wait(b):
            pltpu.make_async_copy(
                rows[b], out_hbm.at[pl.ds(0, chunk)], wsems[b]).wait()

        for b in range(_NBUF):
            pltpu.async_copy(tbl.at[idx_all.at[b]], rows[b], gsems[b])

        def group(g, carry):
            j0 = g * _NBUF
            for b in range(_NBUF):
                gather_wait(b)
                pltpu.async_copy(
                    rows[b],
                    out_hbm.at[pl.ds((chunk0 + j0 + b) * chunk, chunk)],
                    wsems[b])
            for b in range(_NBUF):
                jn = j0 + b + _NBUF

                @pl.when(jn < n_chunks)
                def _():
                    write_wait(b)
                    pltpu.async_copy(tbl.at[idx_all.at[jn]], rows[b], gsems[b])

            return carry

        lax.fori_loop(0, n_groups, group, 0)

        for b in range(_NBUF):
            write_wait(b)

    return gather_kernel


def kernel(z, nuclare_table, elec_W, ls_W, ls_b):
    Bz, L = z.shape
    num_features = nuclare_table.shape[1]
    B = Bz * L

    fused = _compute_fused_table(nuclare_table, elec_W, ls_W, ls_b)

    info = plsc.get_sparse_core_info()
    gather_kernel = _make_sc_gather(B, num_features, info.num_cores,
                                    info.num_subcores)
    z2d = z.reshape(B // 128, 128).astype(jnp.int32)
    out = gather_kernel(z2d, fused)
    return out.reshape(Bz, L, num_features)


# speedup vs baseline: 1.3967x; 1.0050x over previous
import functools

import numpy as np
import jax
import jax.numpy as jnp
from jax import lax
from jax.experimental import pallas as pl
from jax.experimental.pallas import tpu as pltpu
from jax.experimental.pallas import tpu_sc as plsc

_ELEC_ROWS = np.array(
    [
        [0, 0, 0, 0, 0, 0, 0, 0, 0, 0, 0, 0, 0, 0, 0, 0],
        [0, 1, 0, 0, 0, 0, 0, 0, 0, 0, 0, 0, 0, 0, 0, 0],
        [2, 0, 0, 0, 0, 0, 0, 0, 0, 0, 0, 0, 0, 0, 0, 0],
        [2, 0, 0, 1, 0, 0, 0, 0, 0, 0, 0, 0, 0, 0, 0, 0],
        [2, 0, 2, 0, 0, 0, 0, 0, 0, 0, 0, 0, 0, 0, 0, 0],
        [2, 0, 2, 0, 0, 1, 0, 0, 0, 0, 0, 0, 0, 0, 0, 0],
        [2, 0, 2, 0, 0, 2, 0, 0, 0, 0, 0, 0, 0, 0, 0, 0],
        [2, 0, 2, 0, 0, 3, 0, 0, 0, 0, 0, 0, 0, 0, 0, 0],
        [2, 0, 2, 0, 2, 2, 0, 0, 0, 0, 0, 0, 0, 0, 0, 0],
        [2, 0, 2, 0, 4, 1, 0, 0, 0, 0, 0, 0, 0, 0, 0, 0],
    ],
    dtype=np.float32,
)
_ELEC_NORM = _ELEC_ROWS / _ELEC_ROWS.max()
_VPAD = 16
_ELEC_PAD = np.zeros((_VPAD, 16), dtype=np.float32)
_ELEC_PAD[:10] = _ELEC_NORM


def _fused_table_body(elec_ref, nuc_ref, elec_w_ref, ls_w_ref, ls_b_ref, out_ref):
    elec_emb = jnp.dot(elec_ref[...], elec_w_ref[...],
                       preferred_element_type=jnp.float32)
    h = nuc_ref[...] + elec_emb
    h = jnp.dot(h, ls_w_ref[...], preferred_element_type=jnp.float32)
    h = h + ls_b_ref[...]
    out_ref[...] = h * jax.nn.sigmoid(h)


def _compute_fused_table(nuclare_table, elec_W, ls_W, ls_b):
    vocab, num_features = nuclare_table.shape
    nuc_pad = jnp.zeros((_VPAD, num_features), jnp.float32).at[:vocab].set(nuclare_table)
    elec_pad = jnp.asarray(_ELEC_PAD)
    return pl.pallas_call(
        _fused_table_body,
        out_shape=jax.ShapeDtypeStruct((_VPAD, num_features), jnp.float32),
    )(elec_pad, nuc_pad, elec_W, ls_W, ls_b.reshape(1, num_features))


_NBUF = 5


def _make_sc_gather(B, D, nc, ns, chunk=128):
    nw = nc * ns
    b_per_w = B // nw
    n_chunks = b_per_w // chunk
    n_groups = n_chunks // _NBUF
    assert n_chunks % _NBUF == 0
    mesh = plsc.VectorSubcoreMesh(core_axis_name="c", subcore_axis_name="s")

    @functools.partial(
        pl.kernel,
        mesh=mesh,
        out_type=jax.ShapeDtypeStruct((B, D), jnp.float32),
        scratch_types=(
            [pltpu.VMEM((n_chunks, chunk), jnp.int32)]
            + [pltpu.VMEM((chunk, D), jnp.float32) for _ in range(_NBUF)]
            + [pltpu.VMEM_SHARED((ns, _VPAD, D), jnp.float32)]
            + [pltpu.SemaphoreType.DMA for _ in range(2 * _NBUF + 1)]
        ),
    )
    def gather_kernel(z_hbm, table_hbm, out_hbm, idx_all, *bufs_and_sems):
        rows = bufs_and_sems[:_NBUF]
        spm = bufs_and_sems[_NBUF]
        gsems = bufs_and_sems[_NBUF + 1:2 * _NBUF + 1]
        wsems = bufs_and_sems[2 * _NBUF + 1:3 * _NBUF + 1]
        isem = bufs_and_sems[3 * _NBUF + 1]
        sid = lax.axis_index("s")
        wid = sid * nc + lax.axis_index("c")
        chunk0 = wid * n_chunks
        idx_cp = pltpu.async_copy(
            z_hbm.at[pl.ds(chunk0, n_chunks)], idx_all, isem)
        pltpu.sync_copy(table_hbm, rows[0].at[pl.ds(0, _VPAD)])
        pltpu.sync_copy(rows[0].at[pl.ds(0, _VPAD)], spm.at[sid])
        tbl = spm.at[sid]
        idx_cp.wait()

        def gather_wait(b):
            pltpu.make_async_copy(
                tbl.at[idx_all.at[0]], rows[b], gsems[b]).wait()

        def write_wait(b):
            pltpu.make_async_copy(
                rows[b], out_hbm.at[pl.ds(0, chunk)], wsems[b]).wait()

        for b in range(_NBUF):
            pltpu.async_copy(tbl.at[idx_all.at[b]], rows[b], gsems[b])

        def group(g, carry):
            j0 = g * _NBUF
            for b in range(_NBUF):
                gather_wait(b)
                pltpu.async_copy(
                    rows[b],
                    out_hbm.at[pl.ds((chunk0 + j0 + b) * chunk, chunk)],
                    wsems[b])
            for b in range(_NBUF):
                jn = j0 + b + _NBUF

                @pl.when(jn < n_chunks)
                def _():
                    write_wait(b)
                    pltpu.async_copy(tbl.at[idx_all.at[jn]], rows[b], gsems[b])

            return carry

        lax.fori_loop(0, n_groups, group, 0)

        for b in range(_NBUF):
            write_wait(b)

    return gather_kernel


def kernel(z, nuclare_table, elec_W, ls_W, ls_b):
    Bz, L = z.shape
    num_features = nuclare_table.shape[1]
    B = Bz * L

    fused = _compute_fused_table(nuclare_table, elec_W, ls_W, ls_b)

    info = plsc.get_sparse_core_info()
    gather_kernel = _make_sc_gather(B, num_features, info.num_cores,
                                    info.num_subcores)
    z2d = z.reshape(B // 128, 128).astype(jnp.int32)
    out = gather_kernel(z2d, fused)
    return out.reshape(Bz, L, num_features)


# speedup vs baseline: 1.3972x; 1.0004x over previous
import functools

import numpy as np
import jax
import jax.numpy as jnp
from jax import lax
from jax.experimental import pallas as pl
from jax.experimental.pallas import tpu as pltpu
from jax.experimental.pallas import tpu_sc as plsc

_ELEC_ROWS = np.array(
    [
        [0, 0, 0, 0, 0, 0, 0, 0, 0, 0, 0, 0, 0, 0, 0, 0],
        [0, 1, 0, 0, 0, 0, 0, 0, 0, 0, 0, 0, 0, 0, 0, 0],
        [2, 0, 0, 0, 0, 0, 0, 0, 0, 0, 0, 0, 0, 0, 0, 0],
        [2, 0, 0, 1, 0, 0, 0, 0, 0, 0, 0, 0, 0, 0, 0, 0],
        [2, 0, 2, 0, 0, 0, 0, 0, 0, 0, 0, 0, 0, 0, 0, 0],
        [2, 0, 2, 0, 0, 1, 0, 0, 0, 0, 0, 0, 0, 0, 0, 0],
        [2, 0, 2, 0, 0, 2, 0, 0, 0, 0, 0, 0, 0, 0, 0, 0],
        [2, 0, 2, 0, 0, 3, 0, 0, 0, 0, 0, 0, 0, 0, 0, 0],
        [2, 0, 2, 0, 2, 2, 0, 0, 0, 0, 0, 0, 0, 0, 0, 0],
        [2, 0, 2, 0, 4, 1, 0, 0, 0, 0, 0, 0, 0, 0, 0, 0],
    ],
    dtype=np.float32,
)
_ELEC_NORM = _ELEC_ROWS / _ELEC_ROWS.max()
_VPAD = 16
_ELEC_PAD = np.zeros((_VPAD, 16), dtype=np.float32)
_ELEC_PAD[:10] = _ELEC_NORM


def _fused_table_body(elec_ref, nuc_ref, elec_w_ref, ls_w_ref, ls_b_ref, out_ref):
    elec_emb = jnp.dot(elec_ref[...], elec_w_ref[...],
                       preferred_element_type=jnp.float32)
    vocab = nuc_ref.shape[0]
    nuc_pad = jnp.pad(nuc_ref[...], ((0, _VPAD - vocab), (0, 0)))
    h = nuc_pad + elec_emb
    h = jnp.dot(h, ls_w_ref[...], preferred_element_type=jnp.float32)
    h = h + ls_b_ref[...]
    out_ref[...] = h * jax.nn.sigmoid(h)


def _compute_fused_table(nuclare_table, elec_W, ls_W, ls_b):
    num_features = nuclare_table.shape[1]
    elec_pad = jnp.asarray(_ELEC_PAD)
    return pl.pallas_call(
        _fused_table_body,
        out_shape=jax.ShapeDtypeStruct((_VPAD, num_features), jnp.float32),
    )(elec_pad, nuclare_table, elec_W, ls_W, ls_b.reshape(1, num_features))


_NBUF = 5


def _make_sc_gather(B, D, nc, ns, chunk=128):
    nw = nc * ns
    b_per_w = B // nw
    n_chunks = b_per_w // chunk
    n_groups = n_chunks // _NBUF
    assert n_chunks % _NBUF == 0
    mesh = plsc.VectorSubcoreMesh(core_axis_name="c", subcore_axis_name="s")

    @functools.partial(
        pl.kernel,
        mesh=mesh,
        out_type=jax.ShapeDtypeStruct((B, D), jnp.float32),
        scratch_types=(
            [pltpu.VMEM((n_chunks, chunk), jnp.int32)]
            + [pltpu.VMEM((chunk, D), jnp.float32) for _ in range(_NBUF)]
            + [pltpu.VMEM_SHARED((ns, _VPAD, D), jnp.float32)]
            + [pltpu.SemaphoreType.DMA for _ in range(2 * _NBUF + 1)]
        ),
    )
    def gather_kernel(z_hbm, table_hbm, out_hbm, idx_all, *bufs_and_sems):
        rows = bufs_and_sems[:_NBUF]
        spm = bufs_and_sems[_NBUF]
        gsems = bufs_and_sems[_NBUF + 1:2 * _NBUF + 1]
        wsems = bufs_and_sems[2 * _NBUF + 1:3 * _NBUF + 1]
        isem = bufs_and_sems[3 * _NBUF + 1]
        sid = lax.axis_index("s")
        wid = sid * nc + lax.axis_index("c")
        chunk0 = wid * n_chunks
        idx_cp = pltpu.async_copy(
            z_hbm.at[pl.ds(chunk0, n_chunks)], idx_all, isem)
        pltpu.sync_copy(table_hbm, rows[0].at[pl.ds(0, _VPAD)])
        pltpu.sync_copy(rows[0].at[pl.ds(0, _VPAD)], spm.at[sid])
        tbl = spm.at[sid]
        idx_cp.wait()

        def gather_wait(b):
            pltpu.make_async_copy(
                tbl.at[idx_all.at[0]], rows[b], gsems[b]).wait()

        def write_wait(b):
            pltpu.make_async_copy(
                rows[b], out_hbm.at[pl.ds(0, chunk)], wsems[b]).wait()

        for b in range(_NBUF):
            pltpu.async_copy(tbl.at[idx_all.at[b]], rows[b], gsems[b])

        def group(g, carry):
            j0 = g * _NBUF
            for b in range(_NBUF):
                gather_wait(b)
                pltpu.async_copy(
                    rows[b],
                    out_hbm.at[pl.ds((chunk0 + j0 + b) * chunk, chunk)],
                    wsems[b])
            for b in range(_NBUF):
                jn = j0 + b + _NBUF

                @pl.when(jn < n_chunks)
                def _():
                    write_wait(b)
                    pltpu.async_copy(tbl.at[idx_all.at[jn]], rows[b], gsems[b])

            return carry

        lax.fori_loop(0, n_groups, group, 0)

        for b in range(_NBUF):
            write_wait(b)

    return gather_kernel


def kernel(z, nuclare_table, elec_W, ls_W, ls_b):
    Bz, L = z.shape
    num_features = nuclare_table.shape[1]
    B = Bz * L

    fused = _compute_fused_table(nuclare_table, elec_W, ls_W, ls_b)

    info = plsc.get_sparse_core_info()
    gather_kernel = _make_sc_gather(B, num_features, info.num_cores,
                                    info.num_subcores)
    z2d = z.reshape(B // 128, 128).astype(jnp.int32)
    out = gather_kernel(z2d, fused)
    return out.reshape(Bz, L, num_features)
